# baseline (speedup 1.0000x reference)
import functools

import jax
import jax.numpy as jnp
from jax import lax
from jax.experimental import pallas as pl
from jax.experimental.pallas import tpu as pltpu
from jax.experimental.pallas import tpu_sc as plsc

_INFO = plsc.get_sparse_core_info()
_NC = _INFO.num_cores
_NS = _INFO.num_subcores
_NW = _NC * _NS

_CH = 640
_K = 1
_BLK = _CH * _K


@functools.partial(jax.jit, static_argnames=("n_blocks",))
def _gather_flat(idx, table, n_blocks):
    assert n_blocks % 2 == 0 and n_blocks >= 4
    v, d = table.shape
    b_total = _NW * n_blocks * _BLK
    mesh = plsc.VectorSubcoreMesh(core_axis_name="c", subcore_axis_name="s")

    @functools.partial(
        pl.kernel,
        mesh=mesh,
        out_type=jax.ShapeDtypeStruct((b_total, d), jnp.float32),
        scratch_types=[
            pltpu.VMEM((2, _K, _CH), jnp.int32),
            pltpu.VMEM((2, _BLK, d), jnp.float32),
            pltpu.SemaphoreType.DMA,
            pltpu.SemaphoreType.DMA,
            pltpu.SemaphoreType.DMA,
        ],
        compiler_params=pltpu.CompilerParams(use_tc_tiling_on_sc=False),
    )
    def k(idx_hbm, table_hbm, out_hbm, idx_v, rows_v, gsem0, gsem1, ssem):
        wid = lax.axis_index("s") * _NC + lax.axis_index("c")
        base = wid * (n_blocks * _BLK)
        gsems = (gsem0, gsem1)

        def fire_gathers(b, buf):
            pltpu.sync_copy(idx_hbm.at[wid, b], idx_v.at[buf])
            for j in range(_K):
                pltpu.async_copy(
                    table_hbm.at[idx_v.at[buf, j]],
                    rows_v.at[buf, pl.ds(j * _CH, _CH)],
                    gsems[buf],
                )

        def wait_gathers(buf):
            for j in range(_K):
                pltpu.make_async_copy(
                    table_hbm.at[idx_v.at[buf, j]],
                    rows_v.at[buf, pl.ds(j * _CH, _CH)],
                    gsems[buf],
                ).wait()

        def fire_store(b, buf):
            pltpu.async_copy(
                rows_v.at[buf], out_hbm.at[pl.ds(base + b * _BLK, _BLK)], ssem
            )

        def drain_store(buf):
            pltpu.make_async_copy(
                rows_v.at[buf], out_hbm.at[pl.ds(base, _BLK)], ssem
            ).wait()

        fire_gathers(0, 0)

        def body(g, _):
            b0 = 2 * g

            @pl.when(g >= 1)
            def _():
                drain_store(1)

            fire_gathers(b0 + 1, 1)
            wait_gathers(0)
            fire_store(b0, 0)

            @pl.when(g < n_blocks // 2 - 1)
            def _():
                drain_store(0)
                fire_gathers(b0 + 2, 0)

            wait_gathers(1)
            fire_store(b0 + 1, 1)
            return 0

        lax.fori_loop(0, n_blocks // 2, body, 0)
        drain_store(0)
        drain_store(1)

    return k(idx, table)


def kernel(x, weight):
    b0, s = x.shape
    v, d = weight.shape
    b = b0 * s
    xf = x.reshape(b).astype(jnp.int32)

    per_super = _NW * _BLK
    n_blocks = -(-b // per_super)
    if n_blocks % 2:
        n_blocks += 1
    b_pad = n_blocks * per_super
    if b_pad != b:
        xf = jnp.concatenate([xf, jnp.zeros((b_pad - b,), jnp.int32)])
    idx = xf.reshape(_NW, n_blocks, _K, _CH)

    out = _gather_flat(idx, weight, n_blocks)
    if b_pad != b:
        out = out[:b]
    return out.reshape(b0, s, d)


# speedup vs baseline: 1.2310x; 1.2310x over previous
import functools

import jax
import jax.numpy as jnp
from jax import lax
from jax.experimental import pallas as pl
from jax.experimental.pallas import tpu as pltpu
from jax.experimental.pallas import tpu_sc as plsc

_INFO = plsc.get_sparse_core_info()
_NC = _INFO.num_cores
_NS = _INFO.num_subcores
_NW = _NC * _NS

_CH = 128
_K = 2
_BLK = _CH * _K
_DP = 128


@functools.partial(jax.jit, static_argnames=("n_blocks",))
def _gather_flat(idx, table, n_blocks):
    assert n_blocks % 2 == 0 and n_blocks >= 4
    v, dp = table.shape
    d = 64
    b_total = _NW * n_blocks * _BLK
    mesh = plsc.VectorSubcoreMesh(core_axis_name="c", subcore_axis_name="s")

    @functools.partial(
        pl.kernel,
        mesh=mesh,
        out_type=jax.ShapeDtypeStruct((b_total, dp), jnp.float32),
        scratch_types=[
            pltpu.VMEM((2, _K, _CH), jnp.int32),
            pltpu.VMEM((2, _BLK, dp), jnp.float32),
            pltpu.SemaphoreType.DMA,
            pltpu.SemaphoreType.DMA,
            pltpu.SemaphoreType.DMA,
        ],
        compiler_params=pltpu.CompilerParams(use_tc_tiling_on_sc=False),
    )
    def k(idx_hbm, table_hbm, out_hbm, idx_v, rows_v, gsem0, gsem1, ssem):
        wid = lax.axis_index("s") * _NC + lax.axis_index("c")
        base = wid * (n_blocks * _BLK)
        gsems = (gsem0, gsem1)

        def fire_gathers(b, buf):
            pltpu.sync_copy(idx_hbm.at[wid, b], idx_v.at[buf])
            for j in range(_K):
                pltpu.async_copy(
                    table_hbm.at[idx_v.at[buf, j]],
                    rows_v.at[buf, pl.ds(j * _CH, _CH)],
                    gsems[buf],
                )

        def wait_gathers(buf):
            for j in range(_K):
                pltpu.make_async_copy(
                    table_hbm.at[idx_v.at[buf, j]],
                    rows_v.at[buf, pl.ds(j * _CH, _CH)],
                    gsems[buf],
                ).wait()

        def fire_store(b, buf):
            pltpu.async_copy(
                rows_v.at[buf],
                out_hbm.at[pl.ds(base + b * _BLK, _BLK)],
                ssem,
            )

        def drain_store(buf):
            pltpu.make_async_copy(
                rows_v.at[buf],
                out_hbm.at[pl.ds(base, _BLK)],
                ssem,
            ).wait()

        fire_gathers(0, 0)

        def body(g, _):
            b0 = 2 * g

            @pl.when(g >= 1)
            def _():
                drain_store(1)

            fire_gathers(b0 + 1, 1)
            wait_gathers(0)
            fire_store(b0, 0)

            @pl.when(g < n_blocks // 2 - 1)
            def _():
                drain_store(0)
                fire_gathers(b0 + 2, 0)

            wait_gathers(1)
            fire_store(b0 + 1, 1)
            return 0

        lax.fori_loop(0, n_blocks // 2, body, 0)
        drain_store(0)
        drain_store(1)

    return k(idx, table)


def kernel(x, weight):
    b0, s = x.shape
    v, d = weight.shape
    b = b0 * s
    xf = x.reshape(b).astype(jnp.int32)
    wp = jnp.pad(weight, ((0, 0), (0, _DP - d)))

    per_super = _NW * _BLK
    n_blocks = -(-b // per_super)
    if n_blocks % 2:
        n_blocks += 1
    b_pad = n_blocks * per_super
    if b_pad != b:
        xf = jnp.concatenate([xf, jnp.zeros((b_pad - b,), jnp.int32)])
    idx = xf.reshape(_NW, n_blocks, _K, _CH)

    out = _gather_flat(idx, wp, n_blocks)
    out = out[:b, :d]
    return out.reshape(b0, s, d)


# speedup vs baseline: 1.3804x; 1.1214x over previous
import functools

import jax
import jax.numpy as jnp
from jax import lax
from jax.experimental import pallas as pl
from jax.experimental.pallas import tpu as pltpu
from jax.experimental.pallas import tpu_sc as plsc

_INFO = plsc.get_sparse_core_info()
_NC = _INFO.num_cores
_NS = _INFO.num_subcores
_NW = _NC * _NS

_CH = 128
_K = 2
_BLK = _CH * _K
_DP = 128


@functools.partial(jax.jit, static_argnames=("n_blocks",))
def _gather_flat(idx, table, n_blocks):
    assert n_blocks % 2 == 0 and n_blocks >= 4
    v, dp = table.shape
    d = 64
    b_total = _NW * n_blocks * _BLK
    mesh = plsc.VectorSubcoreMesh(core_axis_name="c", subcore_axis_name="s")

    @functools.partial(
        pl.kernel,
        mesh=mesh,
        out_type=jax.ShapeDtypeStruct((b_total, dp), jnp.float32),
        scratch_types=[
            pltpu.VMEM((2, _K, _CH), jnp.int32),
            pltpu.VMEM((2, _BLK, dp), jnp.float32),
            pltpu.SemaphoreType.DMA,
            pltpu.SemaphoreType.DMA,
            pltpu.SemaphoreType.DMA,
        ],
        compiler_params=pltpu.CompilerParams(use_tc_tiling_on_sc=False),
    )
    def k(idx_hbm, table_hbm, out_hbm, idx_v, rows_v, gsem0, gsem1, ssem):
        wid = lax.axis_index("s") * _NC + lax.axis_index("c")
        base = wid * (n_blocks * _BLK)
        gsems = (gsem0, gsem1)

        def fire_gathers(b, buf):
            pltpu.sync_copy(idx_hbm.at[wid, b], idx_v.at[buf])
            for j in range(_K):
                pltpu.async_copy(
                    table_hbm.at[idx_v.at[buf, j]],
                    rows_v.at[buf, pl.ds(j * _CH, _CH)],
                    gsems[buf],
                )

        def wait_gathers(buf):
            for j in range(_K):
                pltpu.make_async_copy(
                    table_hbm.at[idx_v.at[buf, j]],
                    rows_v.at[buf, pl.ds(j * _CH, _CH)],
                    gsems[buf],
                ).wait()

        def fire_store(b, buf):
            pltpu.async_copy(
                rows_v.at[buf],
                out_hbm.at[pl.ds(base + b * _BLK, _BLK)],
                ssem,
            )

        def drain_store(buf):
            pltpu.make_async_copy(
                rows_v.at[buf],
                out_hbm.at[pl.ds(base, _BLK)],
                ssem,
            ).wait()

        fire_gathers(0, 0)

        def body(g, _):
            b0 = 2 * g

            @pl.when(g >= 1)
            def _():
                drain_store(1)

            fire_gathers(b0 + 1, 1)
            wait_gathers(0)
            fire_store(b0, 0)

            @pl.when(g < n_blocks // 2 - 1)
            def _():
                drain_store(0)
                fire_gathers(b0 + 2, 0)

            wait_gathers(1)
            fire_store(b0 + 1, 1)
            return 0

        lax.fori_loop(0, n_blocks // 2, body, 0)
        drain_store(0)
        drain_store(1)

    return k(idx, table)


def kernel(x, weight):
    b0, s = x.shape
    v, d = weight.shape
    b = b0 * s
    xf = x.reshape(b).astype(jnp.int32)
    eye_pad = jnp.eye(d, _DP, dtype=weight.dtype)
    wp = jax.lax.dot(weight, eye_pad, precision=jax.lax.Precision.HIGHEST)

    per_super = _NW * _BLK
    n_blocks = -(-b // per_super)
    if n_blocks % 2:
        n_blocks += 1
    b_pad = n_blocks * per_super
    if b_pad != b:
        xf = jnp.concatenate([xf, jnp.zeros((b_pad - b,), jnp.int32)])
    idx = xf.reshape(_NW, n_blocks, _K, _CH)

    out = _gather_flat(idx, wp, n_blocks)
    out = out[:b, :d]
    return out.reshape(b0, s, d)
